# baseline (speedup 1.0000x reference)
import functools

import jax
import jax.numpy as jnp
from jax import lax
from jax.experimental import pallas as pl
from jax.experimental.pallas import tpu as pltpu
from jax.experimental.pallas import tpu_sc as plsc

B = 32
D = 8732
C = 81
GT_PER_IMG = 8
NEG_FACTOR = 3
ALPHA = 1.0
TN = B * D
SORT_N = 1 << 19
CHUNK = 8736
TN_PAD = 32 * CHUNK
PAY_WIN = CHUNK + 8
BIG = 1 << 30
NEG_INF = -3.0e38


def _cumsum_1d(x):
    n = x.shape[0]
    idx = lax.iota(jnp.int32, n)
    k = 1
    while k < n:
        x = x + jnp.where(idx >= k, jnp.roll(x, k), 0)
        k *= 2
    return x


def _kernel_a(gts_ref, db_ref, pred_ref, valm_ref, key_ref, pay_ref,
              cidx_ref, scal_ref, forced_ref, negc_ref, npos_ref,
              locs_ref, poss_ref):
    b = pl.program_id(0)
    d_iota = lax.iota(jnp.int32, D)

    db = db_ref[...]
    dx1 = db[0, :] - db[2, :] * 0.5
    dy1 = db[1, :] - db[3, :] * 0.5
    dx2 = db[0, :] + db[2, :] * 0.5
    dy2 = db[1, :] + db[3, :] * 0.5
    darea = db[2, :] * db[3, :]

    @pl.when(b == 0)
    def _init():
        negc_ref[0] = 0
        npos_ref[0] = 0
        locs_ref[0] = 0.0
        poss_ref[0] = 0.0
        best_chunks = []
        for c0 in range(0, B * GT_PER_IMG, 16):
            g = gts_ref[pl.ds(c0, 16), :]
            gx1 = g[:, 1] - g[:, 3] * 0.5
            gy1 = g[:, 2] - g[:, 4] * 0.5
            gx2 = g[:, 1] + g[:, 3] * 0.5
            gy2 = g[:, 2] + g[:, 4] * 0.5
            garea = g[:, 3] * g[:, 4]
            ix1 = jnp.maximum(gx1[:, None], dx1[None, :])
            iy1 = jnp.maximum(gy1[:, None], dy1[None, :])
            ix2 = jnp.minimum(gx2[:, None], dx2[None, :])
            iy2 = jnp.minimum(gy2[:, None], dy2[None, :])
            iw = jnp.clip(ix2 - ix1, 0.0, None)
            ih = jnp.clip(iy2 - iy1, 0.0, None)
            inter = iw * ih
            iou = inter / (garea[:, None] + darea[None, :] - inter + 1e-9)
            best_chunks.append(jnp.argmax(iou, axis=1).astype(jnp.int32))
        best_db = jnp.concatenate(best_chunks)
        for bb in range(B):
            row = jnp.full((D,), -1, jnp.int32)
            for gg in range(GT_PER_IMG):
                g = bb * GT_PER_IMG + gg
                row = jnp.where(d_iota == best_db[g], g, row)
            forced_ref[bb, 0, :] = row

    g8 = gts_ref[pl.ds(8 * b, 8), :]
    gx1 = g8[:, 1] - g8[:, 3] * 0.5
    gy1 = g8[:, 2] - g8[:, 4] * 0.5
    gx2 = g8[:, 1] + g8[:, 3] * 0.5
    gy2 = g8[:, 2] + g8[:, 4] * 0.5
    garea = g8[:, 3] * g8[:, 4]
    ix1 = jnp.maximum(gx1[:, None], dx1[None, :])
    iy1 = jnp.maximum(gy1[:, None], dy1[None, :])
    ix2 = jnp.minimum(gx2[:, None], dx2[None, :])
    iy2 = jnp.minimum(gy2[:, None], dy2[None, :])
    iw = jnp.clip(ix2 - ix1, 0.0, None)
    ih = jnp.clip(iy2 - iy1, 0.0, None)
    inter = iw * ih
    iou8 = inter / (garea[:, None] + darea[None, :] - inter + 1e-9)

    max8 = jnp.max(iou8, axis=0)
    arg8 = jnp.argmax(iou8, axis=0).astype(jnp.int32)

    forced = forced_ref[b, 0, :]

    pos = (max8 > 0.5) | (forced >= 0)
    lg = jnp.where(forced >= 0, forced - 8 * b, arg8)

    cls8 = jnp.argmax(g8[:, 5:86], axis=1).astype(jnp.int32)
    mcx = jnp.zeros((D,), jnp.float32)
    mcy = jnp.zeros((D,), jnp.float32)
    mw = jnp.ones((D,), jnp.float32)
    mh = jnp.ones((D,), jnp.float32)
    for g in range(8):
        sel = lg == g
        mcx = jnp.where(sel, g8[g, 1], mcx)
        mcy = jnp.where(sel, g8[g, 2], mcy)
        mw = jnp.where(sel, g8[g, 3], mw)
        mh = jnp.where(sel, g8[g, 4], mh)

    t0 = (mcx - db[0, :]) / (0.1 * db[2, :])
    t1 = (mcy - db[1, :]) / (0.1 * db[3, :])
    t2 = jnp.log(mw / db[2, :]) / 0.2
    t3 = jnp.log(mh / db[3, :]) / 0.2
    loc_contrib = jnp.float32(0.0)
    for j, enc_j in enumerate((t0, t1, t2, t3)):
        ad = jnp.abs(pred_ref[0, :, j] - enc_j)
        sl1 = jnp.where(ad < 1.0, 0.5 * ad * ad, ad - 0.5)
        loc_contrib += jnp.sum(jnp.where(pos, sl1, 0.0))

    lse_parts, val_parts = [], []
    pos_contrib = jnp.float32(0.0)
    CH = 2184
    for s in range(0, D, CH):
        cs = min(CH, D - s)
        xc = pred_ref[0, pl.ds(s, cs), 4:85]
        m2 = jnp.max(xc, axis=1, keepdims=True)
        s2 = jnp.sum(jnp.exp(xc - m2), axis=1, keepdims=True)
        lse_c = (m2 + jnp.log(s2))[:, 0]
        val_parts.append(lse_c - xc[:, 80])
        c_iota = lax.broadcasted_iota(jnp.int32, (cs, C), 1)
        x_c = jnp.zeros((cs,), jnp.float32)
        lg_c = lg[s:s + cs]
        for g in range(8):
            col_g = jnp.sum(jnp.where(c_iota == cls8[g], xc, 0.0), axis=1)
            x_c = jnp.where(lg_c == g, col_g, x_c)
        pos_contrib += jnp.sum(
            jnp.where(pos[s:s + cs], lse_c - x_c, 0.0))
        lse_parts.append(lse_c)
    val = jnp.concatenate(val_parts)

    negf = ~pos
    nfi = negf.astype(jnp.int32)
    carry = negc_ref[0]
    cum = _cumsum_1d(nfi)
    cidx = carry + cum - 1
    row_neg = cum[D - 1]

    valm_ref[0, 0, :] = jnp.where(negf, val, 0.0)
    key_ref[0, 0, :] = jnp.where(negf, val, NEG_INF)
    pay_ref[0, 0, :] = jnp.where(negf, cidx, BIG)
    cidx_ref[0, 0, :] = jnp.maximum(cidx, 0)

    negc_ref[0] = carry + row_neg
    npos_ref[0] = npos_ref[0] + jnp.sum(pos.astype(jnp.int32))
    locs_ref[0] = locs_ref[0] + loc_contrib
    poss_ref[0] = poss_ref[0] + pos_contrib

    @pl.when(b == B - 1)
    def _fin():
        j_iota = lax.broadcasted_iota(jnp.int32, (1, 8), 1)
        row = jnp.zeros((1, 8), jnp.float32)
        row = jnp.where(j_iota == 0, locs_ref[0], row)
        row = jnp.where(j_iota == 1, poss_ref[0], row)
        row = jnp.where(j_iota == 2, npos_ref[0].astype(jnp.float32), row)
        row = jnp.where(j_iota == 3, negc_ref[0].astype(jnp.float32), row)
        scal_ref[...] = row


def _match_and_losses(predicts, gts, dboxes):
    out_shapes = (
        jax.ShapeDtypeStruct((B, 1, D), jnp.float32),
        jax.ShapeDtypeStruct((B, 1, D), jnp.float32),
        jax.ShapeDtypeStruct((B, 1, D), jnp.int32),
        jax.ShapeDtypeStruct((B, 1, D), jnp.int32),
        jax.ShapeDtypeStruct((1, 8), jnp.float32),
    )
    grid = (B,)
    return pl.pallas_call(
        _kernel_a,
        grid=grid,
        in_specs=[
            pl.BlockSpec((B * GT_PER_IMG, 5 + C), lambda b: (0, 0)),
            pl.BlockSpec((4, D), lambda b: (0, 0)),
            pl.BlockSpec((1, D, 4 + C), lambda b: (b, 0, 0)),
        ],
        out_specs=[
            pl.BlockSpec((1, 1, D), lambda b: (b, 0, 0)),
            pl.BlockSpec((1, 1, D), lambda b: (b, 0, 0)),
            pl.BlockSpec((1, 1, D), lambda b: (b, 0, 0)),
            pl.BlockSpec((1, 1, D), lambda b: (b, 0, 0)),
            pl.BlockSpec((1, 8), lambda b: (0, 0)),
        ],
        out_shape=out_shapes,
        scratch_shapes=[
            pltpu.VMEM((B, 1, D), jnp.int32),
            pltpu.SMEM((1,), jnp.int32),
            pltpu.SMEM((1,), jnp.int32),
            pltpu.SMEM((1,), jnp.float32),
            pltpu.SMEM((1,), jnp.float32),
        ],
    )(gts, dboxes.T, predicts)


ROWS = SORT_N // 128
ROW_BITS = 12
LANE_BITS = 7
TOTAL_BITS = 19


N_STAGES = TOTAL_BITS * (TOTAL_BITS + 1) // 2


def _stage_tables():
    sbits, dbits = [], []
    for pp in range(TOTAL_BITS):
        for sbit in range(pp, -1, -1):
            sbits.append(sbit)
            dbits.append(pp + 1)
    return (jnp.asarray(sbits, jnp.int32), jnp.asarray(dbits, jnp.int32))


def _bitonic_kernel(sbit_ref, dbit_ref, key_ref, pay_ref, out_ref,
                    ks_ref, ps_ref, pk_ref, pp_ref):
    t = pl.program_id(0)

    @pl.when(t == 0)
    def _load():
        ks_ref[...] = key_ref[...]
        ps_ref[...] = pay_ref[...]

    sbit = sbit_ref[t]
    dbit = dbit_ref[t]
    row_iota = lax.broadcasted_iota(jnp.int32, (ROWS, 128), 0)
    lane_iota = lax.broadcasted_iota(jnp.int32, (ROWS, 128), 1)
    i2d = row_iota * 128 + lane_iota
    upper = ((i2d >> sbit) & 1) == 1
    dirbit = ((i2d >> dbit) & 1) == 1

    k = ks_ref[...]
    p = ps_ref[...]

    @pl.when(sbit >= LANE_BITS)
    def _row_partner():
        sh = 1 << (sbit - LANE_BITS)
        pk_ref[...] = jnp.where(upper, pltpu.roll(k, sh, 0),
                                pltpu.roll(k, ROWS - sh, 0))
        pp_ref[...] = jnp.where(upper, pltpu.roll(p, sh, 0),
                                pltpu.roll(p, ROWS - sh, 0))

    @pl.when(sbit < LANE_BITS)
    def _lane_partner():
        sh = 1 << sbit
        pk_ref[...] = jnp.where(upper, pltpu.roll(k, sh, 1),
                                pltpu.roll(k, 128 - sh, 1))
        pp_ref[...] = jnp.where(upper, pltpu.roll(p, sh, 1),
                                pltpu.roll(p, 128 - sh, 1))

    pk = pk_ref[...]
    ppay = pp_ref[...]
    self_larger = (k > pk) | ((k == pk) & (p < ppay))
    want_larger = (~dirbit) ^ upper
    take_self = jnp.logical_not(self_larger ^ want_larger)
    ks_ref[...] = jnp.where(take_self, k, pk)
    ps_ref[...] = jnp.where(take_self, p, ppay)

    @pl.when(t == N_STAGES - 1)
    def _store():
        out_ref[...] = jnp.where(take_self, p, ppay)


def _bitonic_sort_payload(key2d, pay2d):
    sbits, dbits = _stage_tables()
    grid_spec = pltpu.PrefetchScalarGridSpec(
        num_scalar_prefetch=2,
        grid=(N_STAGES,),
        in_specs=[
            pl.BlockSpec((ROWS, 128), lambda t, sb, db: (0, 0)),
            pl.BlockSpec((ROWS, 128), lambda t, sb, db: (0, 0)),
        ],
        out_specs=pl.BlockSpec((ROWS, 128), lambda t, sb, db: (0, 0)),
        scratch_shapes=[
            pltpu.VMEM((ROWS, 128), jnp.float32),
            pltpu.VMEM((ROWS, 128), jnp.int32),
            pltpu.VMEM((ROWS, 128), jnp.float32),
            pltpu.VMEM((ROWS, 128), jnp.int32),
        ],
    )
    return pl.pallas_call(
        _bitonic_kernel,
        grid_spec=grid_spec,
        out_shape=jax.ShapeDtypeStruct((ROWS, 128), jnp.int32),
    )(sbits, dbits, key2d, pay2d)


def _sc_select_sum(valm_pad, cidx_pad, payload_sorted, neg_num_vec):
    mesh = plsc.VectorSubcoreMesh(core_axis_name="c", subcore_axis_name="s")

    @functools.partial(
        pl.kernel,
        out_type=jax.ShapeDtypeStruct((32, 16), jnp.float32),
        mesh=mesh,
        scratch_types=[
            pltpu.VMEM((CHUNK,), jnp.float32),
            pltpu.VMEM((CHUNK,), jnp.int32),
            pltpu.VMEM((CHUNK,), jnp.int32),
            pltpu.VMEM((16,), jnp.int32),
            pltpu.VMEM((16,), jnp.float32),
            pltpu.SemaphoreType.DMA,
        ],
    )
    def body(val_hbm, cidx_hbm, pay_hbm, nn_hbm, out_hbm,
             val_v, cidx_v, pay_v, nn_v, acc_v, sem):
        cid = lax.axis_index("c")
        sid = lax.axis_index("s")
        w = sid * 2 + cid
        base = w * CHUNK
        pltpu.sync_copy(val_hbm.at[pl.ds(base, CHUNK)], val_v)
        pltpu.sync_copy(cidx_hbm.at[pl.ds(base, CHUNK)], cidx_v)
        pltpu.sync_copy(nn_hbm, nn_v)
        pltpu.async_copy(pay_hbm.at[cidx_v], pay_v, sem).wait()
        nn = nn_v[...]

        def step(t, acc):
            gathered = pay_v[pl.ds(t * 16, 16)]
            v = val_v[pl.ds(t * 16, 16)]
            return acc + jnp.where(gathered < nn, v, 0.0)

        acc = lax.fori_loop(0, CHUNK // 16, step,
                            jnp.zeros((16,), jnp.float32))
        acc_v[...] = acc
        pltpu.sync_copy(acc_v, out_hbm.at[w])

    return body(valm_pad, cidx_pad, payload_sorted, neg_num_vec)


def kernel(predicts, gts, dboxes):
    valm, key, pay, cidx, scal = _match_and_losses(predicts, gts, dboxes)

    loc_sum = scal[0, 0]
    pos_sum = scal[0, 1]
    n_pos = scal[0, 2]
    m_neg = scal[0, 3]
    neg_num = jnp.minimum(n_pos * NEG_FACTOR, m_neg).astype(jnp.int32)

    key_flat = key.reshape(-1)
    pay_flat = pay.reshape(-1)
    key_padded = jnp.concatenate(
        [key_flat, jnp.full((SORT_N - TN,), NEG_INF, jnp.float32)])
    pay_padded = jnp.concatenate(
        [pay_flat, jnp.full((SORT_N - TN,), BIG, jnp.int32)])
    pay_sorted = _bitonic_sort_payload(
        key_padded.reshape(ROWS, 128), pay_padded.reshape(ROWS, 128))

    valm_pad = jnp.concatenate(
        [valm.reshape(-1), jnp.zeros((TN_PAD - TN,), jnp.float32)])
    cidx_pad = jnp.concatenate(
        [cidx.reshape(-1), jnp.zeros((TN_PAD - TN,), jnp.int32)])
    nn_vec = jnp.full((16,), neg_num, jnp.int32)

    partials = _sc_select_sum(valm_pad, cidx_pad,
                              pay_sorted.reshape(-1), nn_vec)
    neg_sum = jnp.sum(partials)

    return (pos_sum + neg_sum + ALPHA * loc_sum) / n_pos
